# Initial kernel scaffold; baseline (speedup 1.0000x reference)
import jax
import jax.numpy as jnp
from jax.experimental import pallas as pl


def kernel(class_indices, colors, positions, batch, class_table, Wp1, bp1, Wp2, bp2, Wc1, bc1, Wc2, bc2, Wm, bm, We1, be1, bn_g, bn_b, bn_m, bn_v, We2, be2, Wl1, bl1, Wl2, bl2):
    raise NotImplementedError("write your pallas kernel here")


# speedup vs baseline: 11.0348x; 11.0348x over previous
import functools

import jax
import jax.numpy as jnp
from jax import lax
from jax.experimental import pallas as pl
from jax.experimental.pallas import tpu as pltpu
from jax.experimental.pallas import tpu_sc as plsc

_NEG = float("-inf")
_BIGI = 2 ** 30


def _l2n(x):
    return x / jnp.maximum(jnp.sqrt(jnp.sum(x * x, axis=-1, keepdims=True)), 1e-12)


def _dot(a, b):
    return jnp.dot(a, b, preferred_element_type=jnp.float32)


def _embed_body(nc, d, ci_ref, col_ref, pos_ref, tab_ref, wc1_ref, bc1_ref,
                wc2_ref, bc2_ref, wp1_ref, bp1_ref, wp2_ref, bp2_ref,
                wm_ref, bm_ref, emb_ref, sq_ref):
    rb = ci_ref.shape[0]
    iota = lax.broadcasted_iota(jnp.int32, (rb, nc), 1)
    oh = (iota == ci_ref[...]).astype(jnp.float32)
    ce = _l2n(_dot(oh, tab_ref[...]))
    col = _l2n(_dot(jnp.maximum(_dot(col_ref[...], wc1_ref[...]) + bc1_ref[...], 0.0),
                    wc2_ref[...]) + bc2_ref[...])
    pos = _l2n(_dot(jnp.maximum(_dot(pos_ref[...], wp1_ref[...]) + bp1_ref[...], 0.0),
                    wp2_ref[...]) + bp2_ref[...])
    emb = (_dot(ce, wm_ref[0:d, :]) + _dot(col, wm_ref[d:2 * d, :])
           + _dot(pos, wm_ref[2 * d:3 * d, :]) + bm_ref[...])
    emb_ref[...] = emb
    sq_ref[...] = jnp.sum(emb * emb, axis=1, keepdims=True)


def _top2_block(v, idxg):
    m1 = jnp.max(v, axis=1, keepdims=True)
    i1 = jnp.min(jnp.where(v == m1, idxg, _BIGI), axis=1, keepdims=True)
    v2 = jnp.where(idxg == i1, _NEG, v)
    m2 = jnp.max(v2, axis=1, keepdims=True)
    i2 = jnp.min(jnp.where((v2 == m2) & (idxg != i1), idxg, _BIGI),
                 axis=1, keepdims=True)
    return m1, i1, m2, i2


def _pick(va, ia, vb, ib):
    take_b = vb > va
    return jnp.where(take_b, vb, va), jnp.where(take_b, ib, ia)


def _knn_body(cb, ncb, er_ref, sqr_ref, br_ref, embt_ref, sqc_ref, bc_ref,
              i1_ref, i2_ref):
    rb = er_ref.shape[0]
    er = er_ref[...]
    sqr = sqr_ref[...]
    br = br_ref[...]
    rmin = br_ref[0, 0]
    rmax = br_ref[rb - 1, 0]

    carry = (jnp.full((rb, 1), _NEG, jnp.float32), jnp.zeros((rb, 1), jnp.int32),
             jnp.full((rb, 1), _NEG, jnp.float32), jnp.zeros((rb, 1), jnp.int32))

    for i in range(ncb):
        c0 = i * cb
        cmin = bc_ref[0, c0]
        cmax = bc_ref[0, c0 + cb - 1]
        overlap = jnp.logical_and(cmin <= rmax, cmax >= rmin)

        def _update(carry, c0=c0):
            b1, j1, b2, j2 = carry
            ec = embt_ref[:, c0:c0 + cb]
            d = sqr + sqc_ref[:, c0:c0 + cb] - 2.0 * _dot(er, ec)
            same = br == bc_ref[:, c0:c0 + cb]
            v = jnp.where(same, -d, _NEG)
            idxg = c0 + lax.broadcasted_iota(jnp.int32, (rb, cb), 1)
            m1, i1, m2, i2 = _top2_block(v, idxg)
            nb, nbi = _pick(b1, j1, m1, i1)
            blk_wins = m1 > b1
            sa = jnp.where(blk_wins, b1, b2)
            sai = jnp.where(blk_wins, j1, j2)
            sb = jnp.where(blk_wins, m2, m1)
            sbi = jnp.where(blk_wins, i2, i1)
            ns, nsi = _pick(sa, sai, sb, sbi)
            return nb, nbi, ns, nsi

        carry = lax.cond(overlap, _update, lambda c: c, carry)

    _, j1, _, j2 = carry
    i1_ref[...] = j1
    i2_ref[...] = j2


def _sc_gather(emb, idx_flat, d):
    nw = 32
    ng = idx_flat.shape[0]
    b_per_w = ng // nw
    mesh = plsc.VectorSubcoreMesh(core_axis_name="c", subcore_axis_name="s")

    @functools.partial(
        pl.kernel, mesh=mesh,
        out_type=jax.ShapeDtypeStruct((ng, d), jnp.float32),
        scratch_types=[
            pltpu.VMEM((b_per_w,), jnp.int32),
            pltpu.VMEM((b_per_w, d), jnp.float32),
            pltpu.SemaphoreType.DMA,
        ],
    )
    def k(table_hbm, idx_hbm, out_hbm, idx_v, rows_v, sem):
        wid = lax.axis_index("s") * 2 + lax.axis_index("c")
        base = wid * b_per_w
        pltpu.sync_copy(idx_hbm.at[pl.ds(base, b_per_w)], idx_v)
        pltpu.async_copy(table_hbm.at[idx_v], rows_v, sem).wait()
        pltpu.sync_copy(rows_v, out_hbm.at[pl.ds(base, b_per_w)])

    return k(emb, idx_flat)


def _edge_body(nseg, d, nblocks, xi_ref, xj1_ref, xj2_ref, br_ref, we1_ref,
               be1_ref, bng_ref, bnb_ref, bnm_ref, bnv_ref, we2_ref, be2_ref,
               wl1_ref, bl1_ref, wl2_ref, bl2_ref, out_ref, acc_ref):
    pid = pl.program_id(0)

    @pl.when(pid == 0)
    def _init():
        acc_ref[...] = jnp.full((nseg, d), _NEG, jnp.float32)

    xi = xi_ref[...]
    t = _dot(xi, we1_ref[0:d, :]) + be1_ref[...]
    wb = we1_ref[d:2 * d, :]
    inv = 1.0 / jnp.sqrt(bnv_ref[...] + 1e-5)

    def edge(xj):
        h = jnp.maximum(t + _dot(xj - xi, wb), 0.0)
        h = (h - bnm_ref[...]) * inv * bng_ref[...] + bnb_ref[...]
        return _dot(h, we2_ref[...]) + be2_ref[...]

    node = jnp.maximum(edge(xj1_ref[...]), edge(xj2_ref[...]))

    br = br_ref[...]
    for b in range(nseg):
        vals = jnp.where(br == b, node, _NEG)
        pmax = jnp.max(vals, axis=0, keepdims=True)
        acc_ref[b:b + 1, :] = jnp.maximum(acc_ref[b:b + 1, :], pmax)

    @pl.when(pid == nblocks - 1)
    def _final():
        pooled = acc_ref[...]
        o = _dot(jnp.maximum(_dot(pooled, wl1_ref[...]) + bl1_ref[...], 0.0),
                 wl2_ref[...]) + bl2_ref[...]
        out_ref[...] = _l2n(o)


def kernel(class_indices, colors, positions, batch, class_table, Wp1, bp1,
           Wp2, bp2, Wc1, bc1, Wc2, bc2, Wm, bm, We1, be1, bn_g, bn_b, bn_m,
           bn_v, We2, be2, Wl1, bl1, Wl2, bl2):
    n = class_indices.shape[0]
    nc, d = class_table.shape
    nseg = 16

    npad = ((n + 2047) // 2048) * 2048
    rb = 512
    cb = 1024
    ncb = npad // cb
    nblocks = npad // rb
    pad = npad - n

    ci = jnp.pad(class_indices.astype(jnp.int32), (0, pad)).reshape(npad, 1)
    col = jnp.pad(colors, ((0, pad), (0, 0)))
    pos = jnp.pad(positions, ((0, pad), (0, 0)))
    bat = jnp.pad(batch.astype(jnp.int32), (0, pad), constant_values=nseg)
    br_col = bat.reshape(npad, 1)
    bc_row = bat.reshape(1, npad)

    def row(x):
        return x.reshape(1, -1)

    full = lambda shape: pl.BlockSpec(shape, lambda i: (0, 0))
    rows = lambda w: pl.BlockSpec((rb, w), lambda i: (i, 0))

    emb, sq = pl.pallas_call(
        functools.partial(_embed_body, nc, d),
        grid=(nblocks,),
        in_specs=[rows(1), rows(3), rows(3), full((nc, d)),
                  full((3, 32)), full((1, 32)), full((32, d)), full((1, d)),
                  full((3, 32)), full((1, 32)), full((32, d)), full((1, d)),
                  full((3 * d, d)), full((1, d))],
        out_specs=[rows(d), rows(1)],
        out_shape=[jax.ShapeDtypeStruct((npad, d), jnp.float32),
                   jax.ShapeDtypeStruct((npad, 1), jnp.float32)],
    )(ci, col, pos, class_table, Wc1, row(bc1), Wc2, row(bc2),
      Wp1, row(bp1), Wp2, row(bp2), Wm, row(bm))

    embt = emb.T

    i1, i2 = pl.pallas_call(
        functools.partial(_knn_body, cb, ncb),
        grid=(nblocks,),
        in_specs=[rows(d), rows(1), rows(1),
                  full((d, npad)), full((1, npad)), full((1, npad))],
        out_specs=[rows(1), rows(1)],
        out_shape=[jax.ShapeDtypeStruct((npad, 1), jnp.int32),
                   jax.ShapeDtypeStruct((npad, 1), jnp.int32)],
    )(emb, sq, br_col, embt, sq.reshape(1, npad), bc_row)

    ng = ((2 * n + 255) // 256) * 256
    idx_flat = jnp.concatenate([i1[:n, 0], i2[:n, 0],
                                jnp.zeros((ng - 2 * n,), jnp.int32)])
    xj = _sc_gather(emb, idx_flat, d)
    xj1 = jnp.pad(xj[:n], ((0, pad), (0, 0)))
    xj2 = jnp.pad(xj[n:2 * n], ((0, pad), (0, 0)))

    out = pl.pallas_call(
        functools.partial(_edge_body, nseg, d, nblocks),
        grid=(nblocks,),
        in_specs=[rows(d), rows(d), rows(d), rows(1),
                  full((2 * d, d)), full((1, d)), full((1, d)), full((1, d)),
                  full((1, d)), full((1, d)), full((d, d)), full((1, d)),
                  full((d, d)), full((1, d)), full((d, d)), full((1, d))],
        out_specs=pl.BlockSpec((nseg, d), lambda i: (0, 0)),
        out_shape=jax.ShapeDtypeStruct((nseg, d), jnp.float32),
        scratch_shapes=[pltpu.VMEM((nseg, d), jnp.float32)],
    )(emb, xj1, xj2, br_col, We1, row(be1), row(bn_g), row(bn_b), row(bn_m),
      row(bn_v), We2, row(be2), Wl1, row(bl1), Wl2, row(bl2))

    return out


# speedup vs baseline: 11.7468x; 1.0645x over previous
import functools

import jax
import jax.numpy as jnp
from jax import lax
from jax.experimental import pallas as pl
from jax.experimental.pallas import tpu as pltpu
from jax.experimental.pallas import tpu_sc as plsc

_NEG = float("-inf")
_BIGI = 2 ** 30


def _l2n(x):
    return x / jnp.maximum(jnp.sqrt(jnp.sum(x * x, axis=-1, keepdims=True)), 1e-12)


def _dot(a, b, precision=None):
    return jnp.dot(a, b, preferred_element_type=jnp.float32,
                   precision=precision)


def _embed_body(nc, d, ci_ref, col_ref, pos_ref, tab_ref, wc1_ref, bc1_ref,
                wc2_ref, bc2_ref, wp1_ref, bp1_ref, wp2_ref, bp2_ref,
                wm_ref, bm_ref, emb_ref, embt_ref, sqr_ref, sqc_ref):
    rb = ci_ref.shape[0]
    iota = lax.broadcasted_iota(jnp.int32, (rb, nc), 1)
    oh = (iota == ci_ref[...]).astype(jnp.float32)
    ce = _l2n(_dot(oh, tab_ref[...], precision=lax.Precision.HIGHEST))
    col = _l2n(_dot(jnp.maximum(_dot(col_ref[...], wc1_ref[...]) + bc1_ref[...], 0.0),
                    wc2_ref[...]) + bc2_ref[...])
    pos = _l2n(_dot(jnp.maximum(_dot(pos_ref[...], wp1_ref[...]) + bp1_ref[...], 0.0),
                    wp2_ref[...]) + bp2_ref[...])
    emb = _dot(jnp.concatenate([ce, col, pos], axis=1), wm_ref[...]) + bm_ref[...]
    emb_ref[...] = emb
    embt_ref[...] = emb.T
    sq = jnp.sum(emb * emb, axis=1, keepdims=True)
    sqr_ref[...] = sq
    sqc_ref[...] = sq.T


def _top2_block(v, idxg):
    m1 = jnp.max(v, axis=1, keepdims=True)
    i1 = jnp.min(jnp.where(v == m1, idxg, _BIGI), axis=1, keepdims=True)
    v2 = jnp.where(idxg == i1, _NEG, v)
    m2 = jnp.max(v2, axis=1, keepdims=True)
    i2 = jnp.min(jnp.where((v2 == m2) & (idxg != i1), idxg, _BIGI),
                 axis=1, keepdims=True)
    return m1, i1, m2, i2


def _pick(va, ia, vb, ib):
    take_b = vb > va
    return jnp.where(take_b, vb, va), jnp.where(take_b, ib, ia)


def _knn_body(cb, ncb, er_ref, sqr_ref, br_ref, embt_ref, sqc_ref, bc_ref,
              i1_ref, i2_ref):
    rb = er_ref.shape[0]
    er = er_ref[...]
    sqr = sqr_ref[...]
    br = br_ref[...]
    rmin = br_ref[0, 0]
    rmax = br_ref[rb - 1, 0]

    carry = (jnp.full((rb, 1), _NEG, jnp.float32), jnp.zeros((rb, 1), jnp.int32),
             jnp.full((rb, 1), _NEG, jnp.float32), jnp.zeros((rb, 1), jnp.int32))

    for i in range(ncb):
        c0 = i * cb
        cmin = bc_ref[0, c0]
        cmax = bc_ref[0, c0 + cb - 1]
        overlap = jnp.logical_and(cmin <= rmax, cmax >= rmin)

        def _update(carry, c0=c0):
            b1, j1, b2, j2 = carry
            ec = embt_ref[:, c0:c0 + cb]
            d = sqr + sqc_ref[:, c0:c0 + cb] - 2.0 * _dot(er, ec)
            same = br == bc_ref[:, c0:c0 + cb]
            v = jnp.where(same, -d, _NEG)
            idxg = c0 + lax.broadcasted_iota(jnp.int32, (rb, cb), 1)
            m1, i1, m2, i2 = _top2_block(v, idxg)
            nb, nbi = _pick(b1, j1, m1, i1)
            blk_wins = m1 > b1
            sa = jnp.where(blk_wins, b1, b2)
            sai = jnp.where(blk_wins, j1, j2)
            sb = jnp.where(blk_wins, m2, m1)
            sbi = jnp.where(blk_wins, i2, i1)
            ns, nsi = _pick(sa, sai, sb, sbi)
            return nb, nbi, ns, nsi

        carry = lax.cond(overlap, _update, lambda c: c, carry)

    _, j1, _, j2 = carry
    i1_ref[...] = j1
    i2_ref[...] = j2


def _sc_gather(emb, idx_flat, d):
    nw = 32
    ng = idx_flat.shape[0]
    b_per_w = ng // nw
    mesh = plsc.VectorSubcoreMesh(core_axis_name="c", subcore_axis_name="s")

    @functools.partial(
        pl.kernel, mesh=mesh,
        out_type=jax.ShapeDtypeStruct((ng, d), jnp.float32),
        scratch_types=[
            pltpu.VMEM((b_per_w,), jnp.int32),
            pltpu.VMEM((b_per_w, d), jnp.float32),
            pltpu.SemaphoreType.DMA,
        ],
    )
    def k(table_hbm, idx_hbm, out_hbm, idx_v, rows_v, sem):
        wid = lax.axis_index("s") * 2 + lax.axis_index("c")
        base = wid * b_per_w
        pltpu.sync_copy(idx_hbm.at[pl.ds(base, b_per_w)], idx_v)
        pltpu.async_copy(table_hbm.at[idx_v], rows_v, sem).wait()
        pltpu.sync_copy(rows_v, out_hbm.at[pl.ds(base, b_per_w)])

    return k(emb, idx_flat)


def _edge_body(nseg, d, nblocks, xi_ref, xj1_ref, xj2_ref, br_ref, we1_ref,
               be1_ref, bng_ref, bnb_ref, bnm_ref, bnv_ref, we2_ref, be2_ref,
               wl1_ref, bl1_ref, wl2_ref, bl2_ref, out_ref, acc_ref):
    pid = pl.program_id(0)

    @pl.when(pid == 0)
    def _init():
        acc_ref[...] = jnp.full((nseg, d), _NEG, jnp.float32)

    xi = xi_ref[...]
    inv = 1.0 / jnp.sqrt(bnv_ref[...] + 1e-5)

    def edge(xj):
        feat = jnp.concatenate([xi, xj - xi], axis=1)
        h = jnp.maximum(_dot(feat, we1_ref[...]) + be1_ref[...], 0.0)
        h = (h - bnm_ref[...]) * inv * bng_ref[...] + bnb_ref[...]
        return _dot(h, we2_ref[...]) + be2_ref[...]

    node = jnp.maximum(edge(xj1_ref[...]), edge(xj2_ref[...]))

    br = br_ref[...]
    for b in range(nseg):
        vals = jnp.where(br == b, node, _NEG)
        pmax = jnp.max(vals, axis=0, keepdims=True)
        acc_ref[b:b + 1, :] = jnp.maximum(acc_ref[b:b + 1, :], pmax)

    @pl.when(pid == nblocks - 1)
    def _final():
        pooled = acc_ref[...]
        o = _dot(jnp.maximum(_dot(pooled, wl1_ref[...]) + bl1_ref[...], 0.0),
                 wl2_ref[...]) + bl2_ref[...]
        out_ref[...] = _l2n(o)


def kernel(class_indices, colors, positions, batch, class_table, Wp1, bp1,
           Wp2, bp2, Wc1, bc1, Wc2, bc2, Wm, bm, We1, be1, bn_g, bn_b, bn_m,
           bn_v, We2, be2, Wl1, bl1, Wl2, bl2):
    n = class_indices.shape[0]
    nc, d = class_table.shape
    nseg = 16

    npad = ((n + 2047) // 2048) * 2048
    rb = 512
    cb = 1024
    ncb = npad // cb
    nblocks = npad // rb
    pad = npad - n

    ci = jnp.pad(class_indices.astype(jnp.int32), (0, pad)).reshape(npad, 1)
    col = jnp.pad(colors, ((0, pad), (0, 0)))
    pos = jnp.pad(positions, ((0, pad), (0, 0)))
    bat = jnp.pad(batch.astype(jnp.int32), (0, pad), constant_values=nseg)
    br_col = bat.reshape(npad, 1)
    bc_row = bat.reshape(1, npad)

    def row(x):
        return x.reshape(1, -1)

    full = lambda shape: pl.BlockSpec(shape, lambda i: (0, 0))
    rows = lambda w: pl.BlockSpec((rb, w), lambda i: (i, 0))

    cols = lambda w: pl.BlockSpec((w, rb), lambda i: (0, i))

    emb, embt, sq, sqc = pl.pallas_call(
        functools.partial(_embed_body, nc, d),
        grid=(nblocks,),
        in_specs=[rows(1), rows(3), rows(3), full((nc, d)),
                  full((3, 32)), full((1, 32)), full((32, d)), full((1, d)),
                  full((3, 32)), full((1, 32)), full((32, d)), full((1, d)),
                  full((3 * d, d)), full((1, d))],
        out_specs=[rows(d), cols(d), rows(1), cols(1)],
        out_shape=[jax.ShapeDtypeStruct((npad, d), jnp.float32),
                   jax.ShapeDtypeStruct((d, npad), jnp.float32),
                   jax.ShapeDtypeStruct((npad, 1), jnp.float32),
                   jax.ShapeDtypeStruct((1, npad), jnp.float32)],
    )(ci, col, pos, class_table, Wc1, row(bc1), Wc2, row(bc2),
      Wp1, row(bp1), Wp2, row(bp2), Wm, row(bm))

    i1, i2 = pl.pallas_call(
        functools.partial(_knn_body, cb, ncb),
        grid=(nblocks,),
        in_specs=[rows(d), rows(1), rows(1),
                  full((d, npad)), full((1, npad)), full((1, npad))],
        out_specs=[rows(1), rows(1)],
        out_shape=[jax.ShapeDtypeStruct((npad, 1), jnp.int32),
                   jax.ShapeDtypeStruct((npad, 1), jnp.int32)],
    )(emb, sq, br_col, embt, sqc, bc_row)

    idx_flat = jnp.concatenate([i1[:, 0], i2[:, 0]])
    xj = _sc_gather(emb, idx_flat, d)

    out = pl.pallas_call(
        functools.partial(_edge_body, nseg, d, nblocks),
        grid=(nblocks,),
        in_specs=[rows(d),
                  pl.BlockSpec((rb, d), lambda i: (i, 0)),
                  pl.BlockSpec((rb, d), lambda i, nb=nblocks: (i + nb, 0)),
                  rows(1),
                  full((2 * d, d)), full((1, d)), full((1, d)), full((1, d)),
                  full((1, d)), full((1, d)), full((d, d)), full((1, d)),
                  full((d, d)), full((1, d)), full((d, d)), full((1, d))],
        out_specs=pl.BlockSpec((nseg, d), lambda i: (0, 0)),
        out_shape=jax.ShapeDtypeStruct((nseg, d), jnp.float32),
        scratch_shapes=[pltpu.VMEM((nseg, d), jnp.float32)],
    )(emb, xj, xj, br_col, We1, row(be1), row(bn_g), row(bn_b), row(bn_m),
      row(bn_v), We2, row(be2), Wl1, row(bl1), Wl2, row(bl2))

    return out
